# baseline (speedup 1.0000x reference)
import functools

import jax
import jax.numpy as jnp
from jax import lax
from jax.experimental import pallas as pl
from jax.experimental.pallas import tpu as pltpu
from jax.experimental.pallas import tpu_sc as plsc

NC = 2
NS = 16
NW = NC * NS


@functools.lru_cache(maxsize=None)
def _make_sc_gather(n_chunks, chunk, embed, vocab, idx_dtype):
    mesh = plsc.VectorSubcoreMesh(core_axis_name="c", subcore_axis_name="s")

    @functools.partial(
        pl.kernel,
        out_type=jax.ShapeDtypeStruct((NW, n_chunks, chunk, embed), jnp.float32),
        mesh=mesh,
        scratch_types=[
            pltpu.VMEM((n_chunks, chunk), idx_dtype),
            pltpu.VMEM((chunk, embed), jnp.float32),
            pltpu.SemaphoreType.DMA,
        ],
        compiler_params=pltpu.CompilerParams(use_tc_tiling_on_sc=False),
    )
    def gather_kernel(idx_hbm, table_hbm, out_hbm, idx_v, rows_v, gsem):
        wid = lax.axis_index("s") * NC + lax.axis_index("c")
        pltpu.sync_copy(idx_hbm.at[wid], idx_v)

        def step(g, carry):
            pltpu.async_copy(table_hbm.at[idx_v.at[g]], rows_v, gsem).wait()
            pltpu.sync_copy(rows_v, out_hbm.at[wid, g])
            return carry

        lax.fori_loop(0, n_chunks, step, 0)

    return gather_kernel


def kernel(x, table):
    b0, b1 = x.shape
    vocab, embed = table.shape
    total = b0 * b1
    chunk = 512
    assert total % (NW * chunk) == 0
    n_chunks = total // (NW * chunk)
    idx = x.reshape(NW, n_chunks, chunk).astype(jnp.int32)
    fn = _make_sc_gather(n_chunks, chunk, embed, vocab, jnp.int32)
    out = fn(idx, table)
    return out.reshape(b0, b1, embed)


# speedup vs baseline: 1.0209x; 1.0209x over previous
import functools

import jax
import jax.numpy as jnp
from jax import lax
from jax.experimental import pallas as pl
from jax.experimental.pallas import tpu as pltpu
from jax.experimental.pallas import tpu_sc as plsc

NC = 2
NS = 16
NW = NC * NS


NBUF = 4


@functools.lru_cache(maxsize=None)
def _make_sc_gather(n_chunks, chunk, embed, vocab, idx_dtype):
    mesh = plsc.VectorSubcoreMesh(core_axis_name="c", subcore_axis_name="s")
    assert n_chunks % NBUF == 0
    n_rounds = n_chunks // NBUF

    @functools.partial(
        pl.kernel,
        out_type=jax.ShapeDtypeStruct((NW, n_chunks, chunk, embed), jnp.float32),
        mesh=mesh,
        scratch_types=[
            pltpu.VMEM((n_chunks, chunk), idx_dtype),
            [pltpu.VMEM((chunk, embed), jnp.float32) for _ in range(NBUF)],
            [pltpu.SemaphoreType.DMA for _ in range(NBUF)],
            [pltpu.SemaphoreType.DMA for _ in range(NBUF)],
        ],
        compiler_params=pltpu.CompilerParams(use_tc_tiling_on_sc=False),
    )
    def gather_kernel(idx_hbm, table_hbm, out_hbm, idx_v, rows, gsem, osem):
        wid = lax.axis_index("s") * NC + lax.axis_index("c")
        pltpu.sync_copy(idx_hbm.at[wid], idx_v)

        for b in range(NBUF):
            pltpu.async_copy(table_hbm.at[idx_v.at[b]], rows[b], gsem[b])

        def round_body(i, carry):
            base = i * NBUF
            for b in range(NBUF):
                g = base + b
                pltpu.make_async_copy(
                    table_hbm.at[idx_v.at[g]], rows[b], gsem[b]).wait()
                pltpu.async_copy(rows[b], out_hbm.at[wid, g], osem[b])
            for b in range(NBUF):
                g2 = base + NBUF + b
                pltpu.make_async_copy(
                    rows[b], out_hbm.at[wid, base + b], osem[b]).wait()

                @pl.when(g2 < n_chunks)
                def _():
                    pltpu.async_copy(
                        table_hbm.at[idx_v.at[g2]], rows[b], gsem[b])

            return carry

        lax.fori_loop(0, n_rounds, round_body, 0)

    return gather_kernel


def kernel(x, table):
    b0, b1 = x.shape
    vocab, embed = table.shape
    total = b0 * b1
    chunk = 256
    assert total % (NW * chunk) == 0
    n_chunks = total // (NW * chunk)
    idx = x.reshape(NW, n_chunks, chunk).astype(jnp.int32)
    fn = _make_sc_gather(n_chunks, chunk, embed, vocab, jnp.int32)
    out = fn(idx, table)
    return out.reshape(b0, b1, embed)


# speedup vs baseline: 1.4561x; 1.4263x over previous
import functools

import jax
import jax.numpy as jnp
from jax import lax
from jax.experimental import pallas as pl
from jax.experimental.pallas import tpu as pltpu
from jax.experimental.pallas import tpu_sc as plsc

NC = 2
NS = 16
NW = NC * NS
NBUF = 4


@functools.lru_cache(maxsize=None)
def _make_sc_gather(n_chunks, chunk, embed, view_rows):
    mesh = plsc.VectorSubcoreMesh(core_axis_name="c", subcore_axis_name="s")
    assert n_chunks % NBUF == 0
    n_rounds = n_chunks // NBUF
    pitch = 2 * embed

    @functools.partial(
        pl.kernel,
        out_type=jax.ShapeDtypeStruct((NW * n_chunks * chunk, pitch), jnp.float32),
        mesh=mesh,
        scratch_types=[
            pltpu.VMEM((n_chunks, chunk), jnp.int32),
            [pltpu.VMEM((chunk, embed), jnp.float32) for _ in range(NBUF)],
            [pltpu.SemaphoreType.DMA for _ in range(NBUF)],
            [pltpu.SemaphoreType.DMA for _ in range(NBUF)],
        ],
        compiler_params=pltpu.CompilerParams(use_tc_tiling_on_sc=False),
    )
    def gather_kernel(idx_hbm, table_hbm, out_hbm, idx_v, rows, gsem, osem):
        wid = lax.axis_index("s") * NC + lax.axis_index("c")
        base_row = wid * (n_chunks * chunk)
        pltpu.sync_copy(idx_hbm.at[wid], idx_v)

        def dst(g):
            return out_hbm.at[pl.ds(base_row + g * chunk, chunk), pl.ds(0, embed)]

        for b in range(NBUF):
            pltpu.async_copy(table_hbm.at[idx_v.at[b]], rows[b], gsem[b])

        def round_body(i, carry):
            base = i * NBUF
            for b in range(NBUF):
                g = base + b
                pltpu.make_async_copy(
                    table_hbm.at[idx_v.at[g]], rows[b], gsem[b]).wait()
                pltpu.async_copy(rows[b], dst(g), osem[b])
            for b in range(NBUF):
                g2 = base + NBUF + b
                pltpu.make_async_copy(rows[b], dst(base + b), osem[b]).wait()

                @pl.when(g2 < n_chunks)
                def _():
                    pltpu.async_copy(
                        table_hbm.at[idx_v.at[g2]], rows[b], gsem[b])

            return carry

        lax.fori_loop(0, n_rounds, round_body, 0)

    return gather_kernel


def kernel(x, table):
    b0, b1 = x.shape
    vocab, embed = table.shape
    total = b0 * b1
    chunk = 128
    assert total % (NW * chunk) == 0
    n_chunks = total // (NW * chunk)
    tview = jnp.pad(table, ((0, 0), (0, embed))).reshape(2 * vocab, embed)
    idx = (x.astype(jnp.int32) * 2).reshape(NW, n_chunks, chunk)
    fn = _make_sc_gather(n_chunks, chunk, embed, 2 * vocab)
    out = fn(idx, tview)
    return out[:, :embed].reshape(b0, b1, embed)


# speedup vs baseline: 1.4562x; 1.0000x over previous
import functools

import jax
import jax.numpy as jnp
from jax import lax
from jax.experimental import pallas as pl
from jax.experimental.pallas import tpu as pltpu
from jax.experimental.pallas import tpu_sc as plsc

NC = 2
NS = 16
NW = NC * NS
NBUF = 4


@functools.lru_cache(maxsize=None)
def _make_sc_gather(n_chunks, chunk, embed, view_rows):
    mesh = plsc.VectorSubcoreMesh(core_axis_name="c", subcore_axis_name="s")
    assert n_chunks % NBUF == 0
    n_rounds = n_chunks // NBUF
    pitch = 2 * embed

    @functools.partial(
        pl.kernel,
        out_type=jax.ShapeDtypeStruct((NW * n_chunks * chunk, pitch), jnp.float32),
        mesh=mesh,
        scratch_types=[
            pltpu.VMEM((n_chunks, chunk), jnp.int32),
            [pltpu.VMEM((chunk, embed), jnp.float32) for _ in range(NBUF)],
            [pltpu.SemaphoreType.DMA for _ in range(NBUF)],
            [pltpu.SemaphoreType.DMA for _ in range(NBUF)],
        ],
        compiler_params=pltpu.CompilerParams(use_tc_tiling_on_sc=False),
    )
    def gather_kernel(idx_hbm, table_hbm, out_hbm, idx_v, rows, gsem, osem):
        wid = lax.axis_index("s") * NC + lax.axis_index("c")
        base_row = wid * (n_chunks * chunk)
        pltpu.sync_copy(idx_hbm.at[wid], idx_v)

        def dst(g):
            return out_hbm.at[pl.ds(base_row + g * chunk, chunk), pl.ds(0, embed)]

        for b in range(NBUF):
            pltpu.async_copy(table_hbm.at[idx_v.at[b]], rows[b], gsem[b])

        def round_body(i, carry):
            base = i * NBUF
            for b in range(NBUF):
                g = base + b
                pltpu.make_async_copy(
                    table_hbm.at[idx_v.at[g]], rows[b], gsem[b]).wait()
                pltpu.async_copy(rows[b], dst(g), osem[b])
            for b in range(NBUF):
                g2 = base + NBUF + b
                pltpu.make_async_copy(rows[b], dst(base + b), osem[b]).wait()

                @pl.when(g2 < n_chunks)
                def _():
                    pltpu.async_copy(
                        table_hbm.at[idx_v.at[g2]], rows[b], gsem[b])

            return carry

        lax.fori_loop(0, n_rounds, round_body, 0)

    return gather_kernel


def kernel(x, table):
    b0, b1 = x.shape
    vocab, embed = table.shape
    total = b0 * b1
    chunk = 256
    assert total % (NW * chunk) == 0
    n_chunks = total // (NW * chunk)
    tview = jnp.pad(table, ((0, 0), (0, embed))).reshape(2 * vocab, embed)
    idx = (x.astype(jnp.int32) * 2).reshape(NW, n_chunks, chunk)
    fn = _make_sc_gather(n_chunks, chunk, embed, 2 * vocab)
    out = fn(idx, tview)
    return out[:, :embed].reshape(b0, b1, embed)


# speedup vs baseline: 1.8183x; 1.2487x over previous
import functools

import jax
import jax.numpy as jnp
from jax import lax
from jax.experimental import layout as jex_layout
from jax.experimental import pallas as pl
from jax.experimental.pallas import tpu as pltpu
from jax.experimental.pallas import tpu_sc as plsc

NC = 2
NS = 16
NW = NC * NS
NBUF = 4


@functools.lru_cache(maxsize=None)
def _make_sc_gather(n_chunks, chunk, embed, view_rows):
    mesh = plsc.VectorSubcoreMesh(core_axis_name="c", subcore_axis_name="s")
    assert n_chunks % NBUF == 0
    n_rounds = n_chunks // NBUF
    pitch = 2 * embed

    @functools.partial(
        pl.kernel,
        out_type=jax.ShapeDtypeStruct((NW * n_chunks * chunk, pitch), jnp.float32),
        mesh=mesh,
        scratch_types=[
            pltpu.VMEM((n_chunks, chunk), jnp.int32),
            [pltpu.VMEM((chunk, embed), jnp.float32) for _ in range(NBUF)],
            [pltpu.SemaphoreType.DMA for _ in range(NBUF)],
            [pltpu.SemaphoreType.DMA for _ in range(NBUF)],
        ],
        compiler_params=pltpu.CompilerParams(use_tc_tiling_on_sc=False),
    )
    def gather_kernel(idx_hbm, table_hbm, out_hbm, idx_v, rows, gsem, osem):
        wid = lax.axis_index("s") * NC + lax.axis_index("c")
        base_row = wid * (n_chunks * chunk)
        pltpu.sync_copy(idx_hbm.at[wid], idx_v)

        def dst(g):
            return out_hbm.at[pl.ds(base_row + g * chunk, chunk), pl.ds(0, embed)]

        for b in range(NBUF):
            pltpu.async_copy(table_hbm.at[idx_v.at[b]], rows[b], gsem[b])

        def round_body(i, carry):
            base = i * NBUF
            for b in range(NBUF):
                g = base + b
                pltpu.make_async_copy(
                    table_hbm.at[idx_v.at[g]], rows[b], gsem[b]).wait()
                pltpu.async_copy(rows[b], dst(g), osem[b])
            for b in range(NBUF):
                g2 = base + NBUF + b
                pltpu.make_async_copy(rows[b], dst(base + b), osem[b]).wait()

                @pl.when(g2 < n_chunks)
                def _():
                    pltpu.async_copy(
                        table_hbm.at[idx_v.at[g2]], rows[b], gsem[b])

            return carry

        lax.fori_loop(0, n_rounds, round_body, 0)

    return gather_kernel


def kernel(x, table):
    b0, b1 = x.shape
    vocab, embed = table.shape
    total = b0 * b1
    chunk = 256
    assert total % (NW * chunk) == 0
    n_chunks = total // (NW * chunk)
    tview = jnp.pad(table, ((0, 0), (0, embed))).reshape(2 * vocab, embed)
    idx = (x.astype(jnp.int32) * 2).reshape(NW, n_chunks, chunk)
    fn = _make_sc_gather(n_chunks, chunk, embed, 2 * vocab)
    out = fn(idx, tview)
    res = out[:, :embed].reshape(b0, b1, embed)
    return jex_layout.with_layout_constraint(
        res, jex_layout.Layout((0, 1, 2)))
